# Initial kernel scaffold; baseline (speedup 1.0000x reference)
import jax
import jax.numpy as jnp
from jax.experimental import pallas as pl


def kernel(inputs, W1, b1, W2, b2, routing_thresholds, usage_counts, specialization_scores):
    raise NotImplementedError("write your pallas kernel here")


# speedup vs baseline: 1.4046x; 1.4046x over previous
import functools

import jax
import jax.numpy as jnp
from jax.experimental import pallas as pl

D = 4096
H = D // 2
E = 8
TEMPERATURE = 1.0
MIN_USAGE_THRESHOLD = 0.01

TM = 256


def _router_block(x_ref, w1t_ref, b1_ref, w2t_ref, b2_ref, thr_ref, uc_ref,
                  ss_ref, out_ref, mask_ref):
    xb = x_ref[...].astype(jnp.bfloat16)
    h = jax.lax.dot(xb, w1t_ref[...],
                    preferred_element_type=jnp.float32)
    h = jnp.maximum(h + b1_ref[...], 0.0)
    logits = jax.lax.dot(h.astype(jnp.bfloat16), w2t_ref[...],
                         preferred_element_type=jnp.float32)
    logits = (logits + b2_ref[...]) / TEMPERATURE

    uc = uc_ref[...]
    ur = uc / jnp.sum(uc)
    ss = ss_ref[...]
    sp = jnp.exp(ss - jnp.max(ss))
    sp = sp / jnp.sum(sp)
    maskf = jnp.where((ur > MIN_USAGE_THRESHOLD) & (sp > 0.05), 1.0, 0.0)
    mask_ref[...] = maskf

    logits = jnp.where(maskf > 0.5, logits, -jnp.inf)
    m = jnp.max(logits, axis=-1, keepdims=True)
    e = jnp.exp(logits - m)
    p = e / jnp.sum(e, axis=-1, keepdims=True)
    p = jnp.where(p > thr_ref[...], p, 0.0)

    iota = jax.lax.broadcasted_iota(jnp.int32, p.shape, 1)
    m1 = jnp.max(p, axis=-1, keepdims=True)
    i1 = jnp.min(jnp.where(p == m1, iota, E), axis=-1, keepdims=True)
    p2 = jnp.where(iota == i1, -1.0, p)
    m2 = jnp.max(p2, axis=-1, keepdims=True)
    i2 = jnp.min(jnp.where(p2 == m2, iota, E), axis=-1, keepdims=True)
    denom = m1 + m2 + 1e-9
    sel = (iota == i1) | (iota == i2)
    out_ref[...] = jnp.where(sel, p / denom, 0.0)


@functools.partial(jax.jit, static_argnames=())
def kernel(inputs, W1, b1, W2, b2, routing_thresholds, usage_counts,
           specialization_scores):
    B, S, d = inputs.shape
    n = B * S
    x = inputs.reshape(n, d)
    w1t = W1.T.astype(jnp.bfloat16)
    w2t = W2.T.astype(jnp.bfloat16)
    grid = (n // TM,)

    out, maskf = pl.pallas_call(
        _router_block,
        grid=grid,
        in_specs=[
            pl.BlockSpec((TM, D), lambda i: (i, 0)),
            pl.BlockSpec((D, H), lambda i: (0, 0)),
            pl.BlockSpec((1, H), lambda i: (0, 0)),
            pl.BlockSpec((H, E), lambda i: (0, 0)),
            pl.BlockSpec((1, E), lambda i: (0, 0)),
            pl.BlockSpec((1, E), lambda i: (0, 0)),
            pl.BlockSpec((1, E), lambda i: (0, 0)),
            pl.BlockSpec((1, E), lambda i: (0, 0)),
        ],
        out_specs=[
            pl.BlockSpec((TM, E), lambda i: (i, 0)),
            pl.BlockSpec((1, E), lambda i: (0, 0)),
        ],
        out_shape=[
            jax.ShapeDtypeStruct((n, E), jnp.float32),
            jax.ShapeDtypeStruct((1, E), jnp.float32),
        ],
    )(x, w1t, b1.reshape(1, H), w2t, b2.reshape(1, E),
      routing_thresholds.reshape(1, E), usage_counts.reshape(1, E),
      specialization_scores.reshape(1, E))

    routing_weights = out.reshape(B, S, E)
    expert_mask = maskf.reshape(E) > 0.5
    return (routing_weights, expert_mask)


# speedup vs baseline: 1.4903x; 1.0610x over previous
import functools

import jax
import jax.numpy as jnp
from jax.experimental import pallas as pl

D = 4096
H = D // 2
E = 8
TEMPERATURE = 1.0
MIN_USAGE_THRESHOLD = 0.01

TM = 512


def _router_block(x_ref, w1t_ref, b1_ref, w2t_ref, b2_ref, thr_ref, uc_ref,
                  ss_ref, out_ref, mask_ref):
    xb = x_ref[...].astype(jnp.bfloat16)
    h = jax.lax.dot(xb, w1t_ref[...],
                    preferred_element_type=jnp.float32)
    h = jnp.maximum(h + b1_ref[...], 0.0)
    logits = jax.lax.dot(h.astype(jnp.bfloat16), w2t_ref[...],
                         preferred_element_type=jnp.float32)
    logits = (logits + b2_ref[...]) / TEMPERATURE

    uc = uc_ref[...]
    ur = uc / jnp.sum(uc)
    ss = ss_ref[...]
    sp = jnp.exp(ss - jnp.max(ss))
    sp = sp / jnp.sum(sp)
    maskf = jnp.where((ur > MIN_USAGE_THRESHOLD) & (sp > 0.05), 1.0, 0.0)
    mask_ref[...] = maskf

    logits = jnp.where(maskf > 0.5, logits, -jnp.inf)
    m = jnp.max(logits, axis=-1, keepdims=True)
    e = jnp.exp(logits - m)
    p = e / jnp.sum(e, axis=-1, keepdims=True)
    p = jnp.where(p > thr_ref[...], p, 0.0)

    iota = jax.lax.broadcasted_iota(jnp.int32, p.shape, 1)
    m1 = jnp.max(p, axis=-1, keepdims=True)
    i1 = jnp.min(jnp.where(p == m1, iota, E), axis=-1, keepdims=True)
    p2 = jnp.where(iota == i1, -1.0, p)
    m2 = jnp.max(p2, axis=-1, keepdims=True)
    i2 = jnp.min(jnp.where(p2 == m2, iota, E), axis=-1, keepdims=True)
    denom = m1 + m2 + 1e-9
    sel = (iota == i1) | (iota == i2)
    out_ref[...] = jnp.where(sel, p / denom, 0.0)


@functools.partial(jax.jit, static_argnames=())
def kernel(inputs, W1, b1, W2, b2, routing_thresholds, usage_counts,
           specialization_scores):
    B, S, d = inputs.shape
    n = B * S
    x = inputs.reshape(n, d)
    w1t = W1.T.astype(jnp.bfloat16)
    w2t = W2.T.astype(jnp.bfloat16)
    grid = (n // TM,)

    out, maskf = pl.pallas_call(
        _router_block,
        grid=grid,
        in_specs=[
            pl.BlockSpec((TM, D), lambda i: (i, 0)),
            pl.BlockSpec((D, H), lambda i: (0, 0)),
            pl.BlockSpec((1, H), lambda i: (0, 0)),
            pl.BlockSpec((H, E), lambda i: (0, 0)),
            pl.BlockSpec((1, E), lambda i: (0, 0)),
            pl.BlockSpec((1, E), lambda i: (0, 0)),
            pl.BlockSpec((1, E), lambda i: (0, 0)),
            pl.BlockSpec((1, E), lambda i: (0, 0)),
        ],
        out_specs=[
            pl.BlockSpec((TM, E), lambda i: (i, 0)),
            pl.BlockSpec((1, E), lambda i: (0, 0)),
        ],
        out_shape=[
            jax.ShapeDtypeStruct((n, E), jnp.float32),
            jax.ShapeDtypeStruct((1, E), jnp.float32),
        ],
    )(x, w1t, b1.reshape(1, H), w2t, b2.reshape(1, E),
      routing_thresholds.reshape(1, E), usage_counts.reshape(1, E),
      specialization_scores.reshape(1, E))

    routing_weights = out.reshape(B, S, E)
    expert_mask = maskf.reshape(E) > 0.5
    return (routing_weights, expert_mask)
